# Initial kernel scaffold; baseline (speedup 1.0000x reference)
import jax
import jax.numpy as jnp
from jax.experimental import pallas as pl


def kernel(pred_s0, pred_s1, pred_s2, targets):
    raise NotImplementedError("write your pallas kernel here")


# speedup vs baseline: 253.6522x; 253.6522x over previous
import functools
import numpy as np
import jax
import jax.numpy as jnp
from jax import lax
from jax.experimental import pallas as pl
from jax.experimental.pallas import tpu as pltpu
from jax.experimental.pallas import tpu_sc as plsc

NUM_CLASSES = 80
IMG_SIZE = 416
_ANCHORS = np.array([[10., 13.], [16., 30.], [33., 23.], [30., 61.],
                     [62., 45.], [59., 119.], [116., 90.], [156., 198.],
                     [373., 326.]], dtype=np.float32)
_MASKS = [[6, 7, 8], [3, 4, 5], [0, 1, 2]]
_STRIDES = [8, 16, 32]
FS = [IMG_SIZE // s for s in _STRIDES]
B, T = 16, 32
M = B * T
NCH = 5 + NUM_CLASSES
AWH = []
for i in range(3):
    a = _ANCHORS[_MASKS[i]] / float(_STRIDES[i])
    AWH.append(([float(x) for x in a[:, 0]], [float(x) for x in a[:, 1]]))

_E1 = np.float32(1.0) - np.float32(1e-7)
EPS_TERM = float(-np.log(_E1))

_f32 = jnp.float32
_i32 = jnp.int32


def _sc_assign_gather(tt, p2d0, p2d1, p2d2):
    mesh = plsc.VectorSubcoreMesh(core_axis_name="c", subcore_axis_name="s")

    def body(tt_hbm, p0_hbm, p1_hbm, p2_hbm, rows_hbm, meta_hbm,
             tvals, cellv, pairv, metav, rowsv, sem):
        cid = lax.axis_index("c")
        sid = lax.axis_index("s")
        wid = sid * 2 + cid

        preds = [p0_hbm, p1_hbm, p2_hbm]
        iota = lax.iota(_i32, 16)

        def do_unit(scale, b):
            fs = FS[scale]
            aw, ah = AWH[scale]
            p_hbm = preds[scale]
            nrows = B * fs * fs * 3

            for f in range(5):
                pltpu.sync_copy(tt_hbm.at[f, pl.ds(b * T, T)],
                                tvals.at[pl.ds(32 * f, 32)])

            cells = []
            pairs = []
            for v in range(2):
                sl = pl.ds(16 * v, 16)
                tx = tvals[pl.ds(0 * 32 + 16 * v, 16)]
                ty = tvals[pl.ds(1 * 32 + 16 * v, 16)]
                tw = tvals[pl.ds(2 * 32 + 16 * v, 16)]
                th = tvals[pl.ds(3 * 32 + 16 * v, 16)]
                tc = tvals[pl.ds(4 * 32 + 16 * v, 16)]
                box0 = tx * float(fs)
                box1 = ty * float(fs)
                box2 = tw * float(fs)
                box3 = th * float(fs)
                ci = tc.astype(_i32)
                best = None
                aidx = None
                for k in range(3):
                    inter = jnp.minimum(box2, aw[k]) * jnp.minimum(box3, ah[k])
                    union = box2 * box3 + aw[k] * ah[k] - inter
                    iou = inter / (union + 1e-6)
                    if k == 0:
                        best = iou
                        aidx = jnp.zeros((16,), _i32)
                    else:
                        upd = iou > best
                        best = jnp.where(upd, iou, best)
                        aidx = jnp.where(upd, k, aidx)
                gxi = jnp.minimum(box0.astype(_i32), fs - 1)
                gyi = jnp.minimum(box1.astype(_i32), fs - 1)
                cell = ((b * fs + gyi) * fs + gxi) * 3 + aidx
                pair = cell * NUM_CLASSES + ci
                cellv[sl] = cell
                pairv[sl] = pair
                cells.append(cell)
                pairs.append(pair)
                awa = jnp.where(aidx == 0, aw[0],
                                jnp.where(aidx == 1, aw[1], aw[2]))
                aha = jnp.where(aidx == 0, ah[0],
                                jnp.where(aidx == 1, ah[1], ah[2]))
                metav[pl.ds(2 * 32 + 16 * v, 16)] = awa
                metav[pl.ds(3 * 32 + 16 * v, 16)] = aha
                metav[pl.ds(4 * 32 + 16 * v, 16)] = gxi.astype(_f32)
                metav[pl.ds(5 * 32 + 16 * v, 16)] = gyi.astype(_f32)
                metav[pl.ds(6 * 32 + 16 * v, 16)] = jnp.zeros((16,), _f32)
                metav[pl.ds(7 * 32 + 16 * v, 16)] = jnp.zeros((16,), _f32)

            tvec0 = iota
            tvec1 = iota + 16
            zero16 = jnp.zeros((16,), _i32)

            def shift_body(s, carry):
                h0, h1, q0, q1 = carry
                t2a = jnp.minimum(tvec0 + s, 31)
                va = (tvec0 + s) <= 31
                t2b = jnp.minimum(tvec1 + s, 31)
                vb = (tvec1 + s) <= 31
                c2a = plsc.load_gather(cellv, [t2a])
                c2b = plsc.load_gather(cellv, [t2b])
                p2a = plsc.load_gather(pairv, [t2a])
                p2b = plsc.load_gather(pairv, [t2b])
                h0 = h0 | jnp.where((c2a == cells[0]) & va, 1, zero16)
                h1 = h1 | jnp.where((c2b == cells[1]) & vb, 1, zero16)
                q0 = q0 | jnp.where((p2a == pairs[0]) & va, 1, zero16)
                q1 = q1 | jnp.where((p2b == pairs[1]) & vb, 1, zero16)
                return (h0, h1, q0, q1)

            h0, h1, q0, q1 = lax.fori_loop(
                1, 32, shift_body, (zero16, zero16, zero16, zero16))
            metav[pl.ds(0, 16)] = jnp.where(h0 == 0, 1.0, 0.0).astype(_f32)
            metav[pl.ds(16, 16)] = jnp.where(h1 == 0, 1.0, 0.0).astype(_f32)
            metav[pl.ds(32, 16)] = jnp.where(q0 == 0, 1.0, 0.0).astype(_f32)
            metav[pl.ds(48, 16)] = jnp.where(q1 == 0, 1.0, 0.0).astype(_f32)

            for t in range(T):
                ct = cells[t // 16][t % 16]
                pltpu.async_copy(p_hbm.at[ct], rowsv.at[t], sem)
            for t in range(T):
                pltpu.make_async_copy(p_hbm.at[0], rowsv.at[0], sem).wait()

            pltpu.sync_copy(rowsv, rows_hbm.at[scale, b])
            pltpu.sync_copy(metav, meta_hbm.at[scale, b])

        @pl.when(wid < 16)
        def _():
            do_unit(0, wid)
            do_unit(2, wid)

        @pl.when(wid >= 16)
        def _():
            do_unit(1, wid - 16)

    fn = pl.kernel(
        body,
        out_type=(jax.ShapeDtypeStruct((3, B, T, NCH), _f32),
                  jax.ShapeDtypeStruct((3, B, 256), _f32)),
        mesh=mesh,
        compiler_params=pltpu.CompilerParams(needs_layout_passes=False),
        scratch_types=[
            pltpu.VMEM((160,), _f32),
            pltpu.VMEM((32,), _i32),
            pltpu.VMEM((32,), _i32),
            pltpu.VMEM((256,), _f32),
            pltpu.VMEM((T, NCH), _f32),
            pltpu.SemaphoreType.DMA,
        ],
    )
    return fn(tt, p2d0, p2d1, p2d2)


def _dense_conf_sum(p2d, nrows, brows):
    nsteps = nrows // brows
    assert nsteps * brows == nrows

    def body(pred_ref, out_ref, acc_ref):
        i = pl.program_id(0)

        @pl.when(i == 0)
        def _():
            acc_ref[0] = 0.0

        s = jnp.float32(0.0)
        for cch in (4, 4 + NCH, 4 + 2 * NCH):
            x = pred_ref[:, cch:cch + 1]
            p = jax.nn.sigmoid(x)
            g = -jnp.log(1.0 - jnp.clip(p, 1e-7, 1.0 - 1e-7))
            s = s + jnp.sum(g)
        acc_ref[0] = acc_ref[0] + s

        @pl.when(i == nsteps - 1)
        def _():
            out_ref[:, :] = acc_ref[0].reshape(1, 1)

    return pl.pallas_call(
        body,
        grid=(nsteps,),
        in_specs=[pl.BlockSpec((brows, 255), lambda i: (i, 0))],
        out_specs=pl.BlockSpec((1, 1), lambda i: (0, 0)),
        out_shape=jax.ShapeDtypeStruct((1, 1), _f32),
        scratch_shapes=[pltpu.SMEM((1,), _f32)],
    )(p2d)


def _combine(rows, meta, tt3, sd0, sd1, sd2):
    def body(rows_ref, meta_ref, tt_ref, s0_ref, s1_ref, s2_ref, out_ref):
        sds = [s0_ref, s1_ref, s2_ref]
        total = jnp.float32(0.0)
        clip = lambda p: jnp.clip(p, 1e-7, 1.0 - 1e-7)
        sig = jax.nn.sigmoid
        for i in range(3):
            fs = FS[i]
            n = float(B * fs * fs * 3)
            r = rows_ref[i]
            m = meta_ref[i]
            win = m[:, 0, :]
            pairw = m[:, 1, :]
            awa = m[:, 2, :]
            aha = m[:, 3, :]
            gxf = m[:, 4, :]
            gyf = m[:, 5, :]
            box0 = tt_ref[0] * fs
            box1 = tt_ref[1] * fs
            box2 = tt_ref[2] * fs
            box3 = tt_ref[3] * fs
            ci = tt_ref[4].astype(_i32)

            dcx = sig(r[:, :, 0]) + gxf - (box0 - gxf)
            dcy = sig(r[:, :, 1]) + gyf - (box1 - gyf)
            dw = jnp.exp(r[:, :, 2]) * awa - jnp.log(box2 / awa + 1e-6)
            dh = jnp.exp(r[:, :, 3]) * aha - jnp.log(box3 / aha + 1e-6)
            coord = 5.0 * jnp.sum(
                (dcx * dcx + dcy * dcy + dw * dw + dh * dh) * win) / n

            pc = sig(r[:, :, 4])
            P = jnp.sum(win)
            obj = (jnp.sum(-jnp.log(clip(pc)) * win) + (n - P) * EPS_TERM) / n
            sdense = sds[i][0, 0]
            noobj = 0.5 * (sdense - jnp.sum(-jnp.log(1.0 - clip(pc)) * win)
                           + P * EPS_TERM) / n

            spc = sig(r[:, :, 5:])
            cellsum = jnp.sum(-jnp.log(1.0 - clip(spc)), axis=2)
            clsidx = lax.broadcasted_iota(_i32, (B, T, NUM_CLASSES), 2)
            oh = clsidx == ci[:, :, None]
            pv = jnp.sum(jnp.where(oh, spc, 0.0), axis=2)
            h = -jnp.log(clip(pv)) + jnp.log(1.0 - clip(pv))
            cls = (jnp.sum(cellsum * win) + jnp.sum(h * pairw)
                   + (n - P) * NUM_CLASSES * EPS_TERM) / (n * NUM_CLASSES)

            total = total + coord + obj + noobj + cls
        out_ref[:, :] = (total / 3.0).reshape(1, 1)

    return pl.pallas_call(
        body,
        out_shape=jax.ShapeDtypeStruct((1, 1), _f32),
    )(rows, meta, tt3, sd0, sd1, sd2)


def kernel(pred_s0, pred_s1, pred_s2, targets):
    tt = jnp.transpose(targets.reshape(M, 5))
    preds = [pred_s0, pred_s1, pred_s2]
    p2d = [p.reshape(B * fs * fs * 3, NCH) for p, fs in zip(preds, FS)]
    rows, meta = _sc_assign_gather(tt, p2d[0], p2d[1], p2d[2])
    meta = meta.reshape(3, B, 8, T)
    pflat = [p.reshape(B * fs * fs, 3 * NCH) for p, fs in zip(preds, FS)]
    sd0 = _dense_conf_sum(pflat[0], B * FS[0] * FS[0], 1664)
    sd1 = _dense_conf_sum(pflat[1], B * FS[1] * FS[1], 1352)
    sd2 = _dense_conf_sum(pflat[2], B * FS[2] * FS[2], 2704)
    tt3 = tt.reshape(5, B, T)
    out = _combine(rows, meta, tt3, sd0, sd1, sd2)
    return out[0, 0]


# speedup vs baseline: 458.1260x; 1.8061x over previous
import functools
import numpy as np
import jax
import jax.numpy as jnp
from jax import lax
from jax.experimental import pallas as pl
from jax.experimental.pallas import tpu as pltpu
from jax.experimental.pallas import tpu_sc as plsc

NUM_CLASSES = 80
IMG_SIZE = 416
_ANCHORS = np.array([[10., 13.], [16., 30.], [33., 23.], [30., 61.],
                     [62., 45.], [59., 119.], [116., 90.], [156., 198.],
                     [373., 326.]], dtype=np.float32)
_MASKS = [[6, 7, 8], [3, 4, 5], [0, 1, 2]]
_STRIDES = [8, 16, 32]
FS = [IMG_SIZE // s for s in _STRIDES]
B, T = 16, 32
M = B * T
NCH = 5 + NUM_CLASSES
CH3 = 3 * NCH
AWH = []
for i in range(3):
    a = _ANCHORS[_MASKS[i]] / float(_STRIDES[i])
    AWH.append(([float(x) for x in a[:, 0]], [float(x) for x in a[:, 1]]))

_E1 = np.float32(1.0) - np.float32(1e-7)
EPS_TERM = float(-np.log(_E1))

_f32 = jnp.float32
_i32 = jnp.int32


def _sc_assign_gather(tt, p4d0, p4d1, p4d2):
    mesh = plsc.VectorSubcoreMesh(core_axis_name="c", subcore_axis_name="s")

    def body(tt_hbm, p0_hbm, p1_hbm, p2_hbm, rows_hbm, meta_hbm,
             tvals, cellv, pairv, metav, rowsv, sem):
        cid = lax.axis_index("c")
        sid = lax.axis_index("s")
        wid = sid * 2 + cid

        preds = [p0_hbm, p1_hbm, p2_hbm]
        iota = lax.iota(_i32, 16)

        def do_unit(scale, b):
            fs = FS[scale]
            aw, ah = AWH[scale]
            p_hbm = preds[scale]

            for f in range(5):
                pltpu.sync_copy(tt_hbm.at[f, pl.ds(b * T, T)],
                                tvals.at[pl.ds(32 * f, 32)])

            cells, pairs, gxs, gys, aas = [], [], [], [], []
            for v in range(2):
                sl = pl.ds(16 * v, 16)
                tx = tvals[pl.ds(0 * 32 + 16 * v, 16)]
                ty = tvals[pl.ds(1 * 32 + 16 * v, 16)]
                tw = tvals[pl.ds(2 * 32 + 16 * v, 16)]
                th = tvals[pl.ds(3 * 32 + 16 * v, 16)]
                tc = tvals[pl.ds(4 * 32 + 16 * v, 16)]
                box0 = tx * float(fs)
                box1 = ty * float(fs)
                box2 = tw * float(fs)
                box3 = th * float(fs)
                ci = tc.astype(_i32)
                best = None
                aidx = None
                for k in range(3):
                    inter = jnp.minimum(box2, aw[k]) * jnp.minimum(box3, ah[k])
                    union = box2 * box3 + aw[k] * ah[k] - inter
                    iou = inter / (union + 1e-6)
                    if k == 0:
                        best = iou
                        aidx = jnp.zeros((16,), _i32)
                    else:
                        upd = iou > best
                        best = jnp.where(upd, iou, best)
                        aidx = jnp.where(upd, k, aidx)
                gxi = jnp.minimum(box0.astype(_i32), fs - 1)
                gyi = jnp.minimum(box1.astype(_i32), fs - 1)
                cell = ((b * fs + gyi) * fs + gxi) * 3 + aidx
                pair = cell * NUM_CLASSES + ci
                cellv[sl] = cell
                pairv[sl] = pair
                cells.append(cell)
                pairs.append(pair)
                gxs.append(gxi)
                gys.append(gyi)
                aas.append(aidx)
                awa = jnp.where(aidx == 0, aw[0],
                                jnp.where(aidx == 1, aw[1], aw[2]))
                aha = jnp.where(aidx == 0, ah[0],
                                jnp.where(aidx == 1, ah[1], ah[2]))
                metav[pl.ds(2 * 32 + 16 * v, 16)] = awa
                metav[pl.ds(3 * 32 + 16 * v, 16)] = aha
                metav[pl.ds(4 * 32 + 16 * v, 16)] = gxi.astype(_f32)
                metav[pl.ds(5 * 32 + 16 * v, 16)] = gyi.astype(_f32)
                metav[pl.ds(6 * 32 + 16 * v, 16)] = aidx.astype(_f32)
                metav[pl.ds(7 * 32 + 16 * v, 16)] = jnp.zeros((16,), _f32)

            for t in range(T):
                gyt = gys[t // 16][t % 16]
                gxt = gxs[t // 16][t % 16]
                pltpu.async_copy(p_hbm.at[b, gyt, gxt], rowsv.at[t], sem)

            tvec0 = iota
            tvec1 = iota + 16
            zero16 = jnp.zeros((16,), _i32)

            def shift_body(s, carry):
                h0, h1, q0, q1 = carry
                t2a = jnp.minimum(tvec0 + s, 31)
                va = (tvec0 + s) <= 31
                t2b = jnp.minimum(tvec1 + s, 31)
                vb = (tvec1 + s) <= 31
                c2a = plsc.load_gather(cellv, [t2a])
                c2b = plsc.load_gather(cellv, [t2b])
                p2a = plsc.load_gather(pairv, [t2a])
                p2b = plsc.load_gather(pairv, [t2b])
                h0 = h0 | jnp.where((c2a == cells[0]) & va, 1, zero16)
                h1 = h1 | jnp.where((c2b == cells[1]) & vb, 1, zero16)
                q0 = q0 | jnp.where((p2a == pairs[0]) & va, 1, zero16)
                q1 = q1 | jnp.where((p2b == pairs[1]) & vb, 1, zero16)
                return (h0, h1, q0, q1)

            h0, h1, q0, q1 = lax.fori_loop(
                1, 32, shift_body, (zero16, zero16, zero16, zero16))
            metav[pl.ds(0, 16)] = jnp.where(h0 == 0, 1.0, 0.0).astype(_f32)
            metav[pl.ds(16, 16)] = jnp.where(h1 == 0, 1.0, 0.0).astype(_f32)
            metav[pl.ds(32, 16)] = jnp.where(q0 == 0, 1.0, 0.0).astype(_f32)
            metav[pl.ds(48, 16)] = jnp.where(q1 == 0, 1.0, 0.0).astype(_f32)

            for t in range(T):
                pltpu.make_async_copy(p_hbm.at[0, 0, 0], rowsv.at[0],
                                      sem).wait()

            pltpu.sync_copy(rowsv, rows_hbm.at[scale, b])
            pltpu.sync_copy(metav, meta_hbm.at[scale, b])

        @pl.when(wid < 16)
        def _():
            do_unit(0, wid)
            do_unit(2, wid)

        @pl.when(wid >= 16)
        def _():
            do_unit(1, wid - 16)

    fn = pl.kernel(
        body,
        out_type=(jax.ShapeDtypeStruct((3, B, T, CH3), _f32),
                  jax.ShapeDtypeStruct((3, B, 256), _f32)),
        mesh=mesh,
        compiler_params=pltpu.CompilerParams(needs_layout_passes=False),
        scratch_types=[
            pltpu.VMEM((160,), _f32),
            pltpu.VMEM((32,), _i32),
            pltpu.VMEM((32,), _i32),
            pltpu.VMEM((256,), _f32),
            pltpu.VMEM((T, CH3), _f32),
            pltpu.SemaphoreType.DMA,
        ],
    )
    return fn(tt, p4d0, p4d1, p4d2)


def _dense_conf_sum(p4d, fs):
    def body(pred_ref, out_ref, acc_ref):
        i = pl.program_id(0)

        @pl.when(i == 0)
        def _():
            acc_ref[0] = 0.0

        s = jnp.float32(0.0)
        for cch in (4, 4 + NCH, 4 + 2 * NCH):
            x = pred_ref[:, :, :, cch:cch + 1]
            p = jax.nn.sigmoid(x)
            g = -jnp.log(1.0 - jnp.clip(p, 1e-7, 1.0 - 1e-7))
            s = s + jnp.sum(g)
        acc_ref[0] = acc_ref[0] + s

        @pl.when(i == B - 1)
        def _():
            out_ref[:, :] = acc_ref[0].reshape(1, 1)

    return pl.pallas_call(
        body,
        grid=(B,),
        in_specs=[pl.BlockSpec((1, fs, fs, CH3), lambda i: (i, 0, 0, 0))],
        out_specs=pl.BlockSpec((1, 1), lambda i: (0, 0)),
        out_shape=jax.ShapeDtypeStruct((1, 1), _f32),
        scratch_shapes=[pltpu.SMEM((1,), _f32)],
    )(p4d)


def _combine(rows, meta, tt3, sd0, sd1, sd2):
    def body(rows_ref, meta_ref, tt_ref, s0_ref, s1_ref, s2_ref, out_ref):
        sds = [s0_ref, s1_ref, s2_ref]
        total = jnp.float32(0.0)
        clip = lambda p: jnp.clip(p, 1e-7, 1.0 - 1e-7)
        sig = jax.nn.sigmoid
        for i in range(3):
            fs = FS[i]
            n = float(B * fs * fs * 3)
            rfull = rows_ref[i]
            m = meta_ref[i]
            win = m[:, 0, :]
            pairw = m[:, 1, :]
            awa = m[:, 2, :]
            aha = m[:, 3, :]
            gxf = m[:, 4, :]
            gyf = m[:, 5, :]
            af = m[:, 6, :]
            a3 = af[:, :, None]
            r = jnp.where(a3 == 0.0, rfull[:, :, 0:NCH],
                          jnp.where(a3 == 1.0, rfull[:, :, NCH:2 * NCH],
                                    rfull[:, :, 2 * NCH:3 * NCH]))
            box0 = tt_ref[0] * fs
            box1 = tt_ref[1] * fs
            box2 = tt_ref[2] * fs
            box3 = tt_ref[3] * fs
            ci = tt_ref[4].astype(_i32)

            dcx = sig(r[:, :, 0]) + gxf - (box0 - gxf)
            dcy = sig(r[:, :, 1]) + gyf - (box1 - gyf)
            dw = jnp.exp(r[:, :, 2]) * awa - jnp.log(box2 / awa + 1e-6)
            dh = jnp.exp(r[:, :, 3]) * aha - jnp.log(box3 / aha + 1e-6)
            coord = 5.0 * jnp.sum(
                (dcx * dcx + dcy * dcy + dw * dw + dh * dh) * win) / n

            pc = sig(r[:, :, 4])
            P = jnp.sum(win)
            obj = (jnp.sum(-jnp.log(clip(pc)) * win) + (n - P) * EPS_TERM) / n
            sdense = sds[i][0, 0]
            noobj = 0.5 * (sdense - jnp.sum(-jnp.log(1.0 - clip(pc)) * win)
                           + P * EPS_TERM) / n

            spc = sig(r[:, :, 5:])
            cellsum = jnp.sum(-jnp.log(1.0 - clip(spc)), axis=2)
            clsidx = lax.broadcasted_iota(_i32, (B, T, NUM_CLASSES), 2)
            oh = clsidx == ci[:, :, None]
            pv = jnp.sum(jnp.where(oh, spc, 0.0), axis=2)
            h = -jnp.log(clip(pv)) + jnp.log(1.0 - clip(pv))
            cls = (jnp.sum(cellsum * win) + jnp.sum(h * pairw)
                   + (n - P) * NUM_CLASSES * EPS_TERM) / (n * NUM_CLASSES)

            total = total + coord + obj + noobj + cls
        out_ref[:, :] = (total / 3.0).reshape(1, 1)

    return pl.pallas_call(
        body,
        out_shape=jax.ShapeDtypeStruct((1, 1), _f32),
    )(rows, meta, tt3, sd0, sd1, sd2)


def kernel(pred_s0, pred_s1, pred_s2, targets):
    tt = jnp.transpose(targets.reshape(M, 5))
    rows, meta = _sc_assign_gather(tt, pred_s0, pred_s1, pred_s2)
    meta = meta.reshape(3, B, 8, T)
    sd0 = _dense_conf_sum(pred_s0, FS[0])
    sd1 = _dense_conf_sum(pred_s1, FS[1])
    sd2 = _dense_conf_sum(pred_s2, FS[2])
    tt3 = tt.reshape(5, B, T)
    out = _combine(rows, meta, tt3, sd0, sd1, sd2)
    return out[0, 0]


# speedup vs baseline: 505.1877x; 1.1027x over previous
import functools
import numpy as np
import jax
import jax.numpy as jnp
from jax import lax
from jax.experimental import pallas as pl
from jax.experimental.pallas import tpu as pltpu
from jax.experimental.pallas import tpu_sc as plsc

NUM_CLASSES = 80
IMG_SIZE = 416
_ANCHORS = np.array([[10., 13.], [16., 30.], [33., 23.], [30., 61.],
                     [62., 45.], [59., 119.], [116., 90.], [156., 198.],
                     [373., 326.]], dtype=np.float32)
_MASKS = [[6, 7, 8], [3, 4, 5], [0, 1, 2]]
_STRIDES = [8, 16, 32]
FS = [IMG_SIZE // s for s in _STRIDES]
B, T = 16, 32
M = B * T
NCH = 5 + NUM_CLASSES
CH3 = 3 * NCH
AWH = []
for i in range(3):
    a = _ANCHORS[_MASKS[i]] / float(_STRIDES[i])
    AWH.append(([float(x) for x in a[:, 0]], [float(x) for x in a[:, 1]]))

_E1 = np.float32(1.0) - np.float32(1e-7)
EPS_TERM = float(-np.log(_E1))

_f32 = jnp.float32
_i32 = jnp.int32


def _sc_assign_gather(tflat, p4d0, p4d1, p4d2):
    mesh = plsc.VectorSubcoreMesh(core_axis_name="c", subcore_axis_name="s")

    def body(t_hbm, p0_hbm, p1_hbm, p2_hbm, rows_hbm, meta_hbm,
             traw, cellv, pairv, metav, rowsv, sem):
        cid = lax.axis_index("c")
        sid = lax.axis_index("s")
        wid = sid * 2 + cid

        preds = [p0_hbm, p1_hbm, p2_hbm]
        iota = lax.iota(_i32, 16)

        def do_unit(scale, b):
            fs = FS[scale]
            aw, ah = AWH[scale]
            p_hbm = preds[scale]

            pltpu.sync_copy(t_hbm.at[pl.ds(b * (T * 5), T * 5)], traw)

            cells, pairs, gxs, gys, aas = [], [], [], [], []
            for v in range(2):
                sl = pl.ds(16 * v, 16)
                fidx = (iota + 16 * v) * 5
                tx = plsc.load_gather(traw, [fidx])
                ty = plsc.load_gather(traw, [fidx + 1])
                tw = plsc.load_gather(traw, [fidx + 2])
                th = plsc.load_gather(traw, [fidx + 3])
                tc = plsc.load_gather(traw, [fidx + 4])
                box0 = tx * float(fs)
                box1 = ty * float(fs)
                box2 = tw * float(fs)
                box3 = th * float(fs)
                ci = tc.astype(_i32)
                best = None
                aidx = None
                for k in range(3):
                    inter = jnp.minimum(box2, aw[k]) * jnp.minimum(box3, ah[k])
                    union = box2 * box3 + aw[k] * ah[k] - inter
                    iou = inter / (union + 1e-6)
                    if k == 0:
                        best = iou
                        aidx = jnp.zeros((16,), _i32)
                    else:
                        upd = iou > best
                        best = jnp.where(upd, iou, best)
                        aidx = jnp.where(upd, k, aidx)
                gxi = jnp.minimum(box0.astype(_i32), fs - 1)
                gyi = jnp.minimum(box1.astype(_i32), fs - 1)
                cell = ((b * fs + gyi) * fs + gxi) * 3 + aidx
                pair = cell * NUM_CLASSES + ci
                cellv[sl] = cell
                pairv[sl] = pair
                cells.append(cell)
                pairs.append(pair)
                gxs.append(gxi)
                gys.append(gyi)
                aas.append(aidx)
                awa = jnp.where(aidx == 0, aw[0],
                                jnp.where(aidx == 1, aw[1], aw[2]))
                aha = jnp.where(aidx == 0, ah[0],
                                jnp.where(aidx == 1, ah[1], ah[2]))
                metav[pl.ds(2 * 32 + 16 * v, 16)] = awa
                metav[pl.ds(3 * 32 + 16 * v, 16)] = aha
                metav[pl.ds(4 * 32 + 16 * v, 16)] = gxi.astype(_f32)
                metav[pl.ds(5 * 32 + 16 * v, 16)] = gyi.astype(_f32)
                metav[pl.ds(6 * 32 + 16 * v, 16)] = aidx.astype(_f32)
                metav[pl.ds(7 * 32 + 16 * v, 16)] = jnp.zeros((16,), _f32)

            for t in range(T):
                gyt = gys[t // 16][t % 16]
                gxt = gxs[t // 16][t % 16]
                pltpu.async_copy(p_hbm.at[b, gyt, gxt], rowsv.at[t], sem)

            tvec0 = iota
            tvec1 = iota + 16
            zero16 = jnp.zeros((16,), _i32)

            def shift_body(s, carry):
                h0, h1, q0, q1 = carry
                t2a = jnp.minimum(tvec0 + s, 31)
                va = (tvec0 + s) <= 31
                t2b = jnp.minimum(tvec1 + s, 31)
                vb = (tvec1 + s) <= 31
                c2a = plsc.load_gather(cellv, [t2a])
                c2b = plsc.load_gather(cellv, [t2b])
                p2a = plsc.load_gather(pairv, [t2a])
                p2b = plsc.load_gather(pairv, [t2b])
                h0 = h0 | jnp.where((c2a == cells[0]) & va, 1, zero16)
                h1 = h1 | jnp.where((c2b == cells[1]) & vb, 1, zero16)
                q0 = q0 | jnp.where((p2a == pairs[0]) & va, 1, zero16)
                q1 = q1 | jnp.where((p2b == pairs[1]) & vb, 1, zero16)
                return (h0, h1, q0, q1)

            h0, h1, q0, q1 = lax.fori_loop(
                1, 32, shift_body, (zero16, zero16, zero16, zero16))
            metav[pl.ds(0, 16)] = jnp.where(h0 == 0, 1.0, 0.0).astype(_f32)
            metav[pl.ds(16, 16)] = jnp.where(h1 == 0, 1.0, 0.0).astype(_f32)
            metav[pl.ds(32, 16)] = jnp.where(q0 == 0, 1.0, 0.0).astype(_f32)
            metav[pl.ds(48, 16)] = jnp.where(q1 == 0, 1.0, 0.0).astype(_f32)

            for t in range(T):
                pltpu.make_async_copy(p_hbm.at[0, 0, 0], rowsv.at[0],
                                      sem).wait()

            pltpu.sync_copy(rowsv, rows_hbm.at[scale, b])
            pltpu.sync_copy(metav, meta_hbm.at[scale, b])

        @pl.when(wid < 16)
        def _():
            do_unit(0, wid)
            do_unit(2, wid)

        @pl.when(wid >= 16)
        def _():
            do_unit(1, wid - 16)

    fn = pl.kernel(
        body,
        out_type=(jax.ShapeDtypeStruct((3, B, T, CH3), _f32),
                  jax.ShapeDtypeStruct((3, B, 256), _f32)),
        mesh=mesh,
        compiler_params=pltpu.CompilerParams(needs_layout_passes=False),
        scratch_types=[
            pltpu.VMEM((T * 5,), _f32),
            pltpu.VMEM((32,), _i32),
            pltpu.VMEM((32,), _i32),
            pltpu.VMEM((256,), _f32),
            pltpu.VMEM((T, CH3), _f32),
            pltpu.SemaphoreType.DMA,
        ],
    )
    return fn(tflat, p4d0, p4d1, p4d2)


def _tc_fused(p0, p1, p2, rows, meta, targets):
    def body(p0_ref, p1_ref, p2_ref, rows_ref, meta_ref, tg_ref, out_ref,
             acc_ref):
        i = pl.program_id(0)

        @pl.when(i == 0)
        def _():
            for s in range(3):
                acc_ref[s] = 0.0

        clip = lambda p: jnp.clip(p, 1e-7, 1.0 - 1e-7)
        sig = jax.nn.sigmoid

        for s, pref in enumerate((p0_ref, p1_ref, p2_ref)):
            tot = jnp.float32(0.0)
            for cch in (4, 4 + NCH, 4 + 2 * NCH):
                x = pref[:, :, :, cch:cch + 1]
                p = sig(x)
                g = -jnp.log(1.0 - clip(p))
                tot = tot + jnp.sum(g)
            acc_ref[s] = acc_ref[s] + tot

        @pl.when(i == B - 1)
        def _():
            tg = tg_ref[...]
            total = jnp.float32(0.0)
            for s in range(3):
                fs = FS[s]
                n = float(B * fs * fs * 3)
                rfull = rows_ref[s]
                m = meta_ref[s]
                win = m[:, 0, :]
                pairw = m[:, 1, :]
                awa = m[:, 2, :]
                aha = m[:, 3, :]
                gxf = m[:, 4, :]
                gyf = m[:, 5, :]
                af = m[:, 6, :]
                a3 = af[:, :, None]
                r = jnp.where(a3 == 0.0, rfull[:, :, 0:NCH],
                              jnp.where(a3 == 1.0, rfull[:, :, NCH:2 * NCH],
                                        rfull[:, :, 2 * NCH:3 * NCH]))
                box0 = tg[:, :, 0] * fs
                box1 = tg[:, :, 1] * fs
                box2 = tg[:, :, 2] * fs
                box3 = tg[:, :, 3] * fs
                ci = tg[:, :, 4].astype(_i32)

                dcx = sig(r[:, :, 0]) + gxf - (box0 - gxf)
                dcy = sig(r[:, :, 1]) + gyf - (box1 - gyf)
                dw = jnp.exp(r[:, :, 2]) * awa - jnp.log(box2 / awa + 1e-6)
                dh = jnp.exp(r[:, :, 3]) * aha - jnp.log(box3 / aha + 1e-6)
                coord = 5.0 * jnp.sum(
                    (dcx * dcx + dcy * dcy + dw * dw + dh * dh) * win) / n

                pc = sig(r[:, :, 4])
                P = jnp.sum(win)
                obj = (jnp.sum(-jnp.log(clip(pc)) * win)
                       + (n - P) * EPS_TERM) / n
                noobj = 0.5 * (acc_ref[s]
                               - jnp.sum(-jnp.log(1.0 - clip(pc)) * win)
                               + P * EPS_TERM) / n

                spc = sig(r[:, :, 5:])
                cellsum = jnp.sum(-jnp.log(1.0 - clip(spc)), axis=2)
                clsidx = lax.broadcasted_iota(_i32, (B, T, NUM_CLASSES), 2)
                oh = clsidx == ci[:, :, None]
                pv = jnp.sum(jnp.where(oh, spc, 0.0), axis=2)
                h = -jnp.log(clip(pv)) + jnp.log(1.0 - clip(pv))
                cls = (jnp.sum(cellsum * win) + jnp.sum(h * pairw)
                       + (n - P) * NUM_CLASSES * EPS_TERM) / (n * NUM_CLASSES)

                total = total + coord + obj + noobj + cls
            out_ref[:, :] = (total / 3.0).reshape(1, 1)

    z4 = lambda i: (i, 0, 0, 0)
    park3 = lambda i: (0, 0, 0)
    return pl.pallas_call(
        body,
        grid=(B,),
        in_specs=[
            pl.BlockSpec((1, FS[0], FS[0], CH3), z4),
            pl.BlockSpec((1, FS[1], FS[1], CH3), z4),
            pl.BlockSpec((1, FS[2], FS[2], CH3), z4),
            pl.BlockSpec((3, B, T, CH3), lambda i: (0, 0, 0, 0)),
            pl.BlockSpec((3, B, 8, T), lambda i: (0, 0, 0, 0)),
            pl.BlockSpec((B, T, 5), park3),
        ],
        out_specs=pl.BlockSpec((1, 1), lambda i: (0, 0)),
        out_shape=jax.ShapeDtypeStruct((1, 1), _f32),
        scratch_shapes=[pltpu.SMEM((3,), _f32)],
    )(p0, p1, p2, rows, meta, targets)


def kernel(pred_s0, pred_s1, pred_s2, targets):
    tflat = targets.reshape(M * 5)
    rows, meta = _sc_assign_gather(tflat, pred_s0, pred_s1, pred_s2)
    meta = meta.reshape(3, B, 8, T)
    out = _tc_fused(pred_s0, pred_s1, pred_s2, rows, meta, targets)
    return out[0, 0]


# speedup vs baseline: 573.7742x; 1.1358x over previous
import functools
import numpy as np
import jax
import jax.numpy as jnp
from jax import lax
from jax.experimental import pallas as pl
from jax.experimental.pallas import tpu as pltpu
from jax.experimental.pallas import tpu_sc as plsc

NUM_CLASSES = 80
IMG_SIZE = 416
_ANCHORS = np.array([[10., 13.], [16., 30.], [33., 23.], [30., 61.],
                     [62., 45.], [59., 119.], [116., 90.], [156., 198.],
                     [373., 326.]], dtype=np.float32)
_MASKS = [[6, 7, 8], [3, 4, 5], [0, 1, 2]]
_STRIDES = [8, 16, 32]
FS = [IMG_SIZE // s for s in _STRIDES]
B, T = 16, 32
M = B * T
NCH = 5 + NUM_CLASSES
CH3 = 3 * NCH
AWH = []
for i in range(3):
    a = _ANCHORS[_MASKS[i]] / float(_STRIDES[i])
    AWH.append(([float(x) for x in a[:, 0]], [float(x) for x in a[:, 1]]))

_E1 = np.float32(1.0) - np.float32(1e-7)
EPS_TERM = float(-np.log(_E1))

_f32 = jnp.float32
_i32 = jnp.int32


def _sc_assign_gather(tflat, p4d0, p4d1, p4d2):
    mesh = plsc.VectorSubcoreMesh(core_axis_name="c", subcore_axis_name="s")

    def body(t_hbm, p0_hbm, p1_hbm, p2_hbm, rows_hbm, meta_hbm,
             traw, cellv, pairv, metav, rowsv, sem):
        cid = lax.axis_index("c")
        sid = lax.axis_index("s")
        wid = sid * 2 + cid

        preds = [p0_hbm, p1_hbm, p2_hbm]
        iota = lax.iota(_i32, 16)

        def do_unit(scale, b):
            fs = FS[scale]
            aw, ah = AWH[scale]
            p_hbm = preds[scale]

            pltpu.sync_copy(t_hbm.at[pl.ds(b * (T * 5), T * 5)], traw)

            cells, pairs, gxs, gys, aas = [], [], [], [], []
            for v in range(2):
                sl = pl.ds(16 * v, 16)
                fidx = (iota + 16 * v) * 5
                tx = plsc.load_gather(traw, [fidx])
                ty = plsc.load_gather(traw, [fidx + 1])
                tw = plsc.load_gather(traw, [fidx + 2])
                th = plsc.load_gather(traw, [fidx + 3])
                tc = plsc.load_gather(traw, [fidx + 4])
                box0 = tx * float(fs)
                box1 = ty * float(fs)
                box2 = tw * float(fs)
                box3 = th * float(fs)
                ci = tc.astype(_i32)
                best = None
                aidx = None
                for k in range(3):
                    inter = jnp.minimum(box2, aw[k]) * jnp.minimum(box3, ah[k])
                    union = box2 * box3 + aw[k] * ah[k] - inter
                    iou = inter / (union + 1e-6)
                    if k == 0:
                        best = iou
                        aidx = jnp.zeros((16,), _i32)
                    else:
                        upd = iou > best
                        best = jnp.where(upd, iou, best)
                        aidx = jnp.where(upd, k, aidx)
                gxi = jnp.minimum(box0.astype(_i32), fs - 1)
                gyi = jnp.minimum(box1.astype(_i32), fs - 1)
                cell = ((b * fs + gyi) * fs + gxi) * 3 + aidx
                pair = cell * NUM_CLASSES + ci
                cellv[sl] = cell
                pairv[sl] = pair
                cells.append(cell)
                pairs.append(pair)
                gxs.append(gxi)
                gys.append(gyi)
                aas.append(aidx)
                awa = jnp.where(aidx == 0, aw[0],
                                jnp.where(aidx == 1, aw[1], aw[2]))
                aha = jnp.where(aidx == 0, ah[0],
                                jnp.where(aidx == 1, ah[1], ah[2]))
                metav[pl.ds(2 * 32 + 16 * v, 16)] = awa
                metav[pl.ds(3 * 32 + 16 * v, 16)] = aha
                metav[pl.ds(4 * 32 + 16 * v, 16)] = gxi.astype(_f32)
                metav[pl.ds(5 * 32 + 16 * v, 16)] = gyi.astype(_f32)
                metav[pl.ds(6 * 32 + 16 * v, 16)] = aidx.astype(_f32)
                metav[pl.ds(7 * 32 + 16 * v, 16)] = jnp.zeros((16,), _f32)

            for t in range(T):
                gyt = gys[t // 16][t % 16]
                gxt = gxs[t // 16][t % 16]
                pltpu.async_copy(p_hbm.at[b, gyt, gxt], rowsv.at[t], sem)

            tvec0 = iota
            tvec1 = iota + 16
            zero16 = jnp.zeros((16,), _i32)

            def shift_body(s, carry):
                h0, h1, q0, q1 = carry
                t2a = jnp.minimum(tvec0 + s, 31)
                va = (tvec0 + s) <= 31
                t2b = jnp.minimum(tvec1 + s, 31)
                vb = (tvec1 + s) <= 31
                c2a = plsc.load_gather(cellv, [t2a])
                c2b = plsc.load_gather(cellv, [t2b])
                p2a = plsc.load_gather(pairv, [t2a])
                p2b = plsc.load_gather(pairv, [t2b])
                h0 = h0 | jnp.where((c2a == cells[0]) & va, 1, zero16)
                h1 = h1 | jnp.where((c2b == cells[1]) & vb, 1, zero16)
                q0 = q0 | jnp.where((p2a == pairs[0]) & va, 1, zero16)
                q1 = q1 | jnp.where((p2b == pairs[1]) & vb, 1, zero16)
                return (h0, h1, q0, q1)

            h0, h1, q0, q1 = lax.fori_loop(
                1, 32, shift_body, (zero16, zero16, zero16, zero16))
            metav[pl.ds(0, 16)] = jnp.where(h0 == 0, 1.0, 0.0).astype(_f32)
            metav[pl.ds(16, 16)] = jnp.where(h1 == 0, 1.0, 0.0).astype(_f32)
            metav[pl.ds(32, 16)] = jnp.where(q0 == 0, 1.0, 0.0).astype(_f32)
            metav[pl.ds(48, 16)] = jnp.where(q1 == 0, 1.0, 0.0).astype(_f32)

            for t in range(T):
                pltpu.make_async_copy(p_hbm.at[0, 0, 0], rowsv.at[0],
                                      sem).wait()

            pltpu.sync_copy(rowsv, rows_hbm.at[scale, b])
            pltpu.sync_copy(metav, meta_hbm.at[scale, b])

        @pl.when(wid < 16)
        def _():
            do_unit(0, wid)
            do_unit(2, wid)

        @pl.when(wid >= 16)
        def _():
            do_unit(1, wid - 16)

    fn = pl.kernel(
        body,
        out_type=(jax.ShapeDtypeStruct((3, B, T, CH3), _f32),
                  jax.ShapeDtypeStruct((3, B, 256), _f32)),
        mesh=mesh,
        compiler_params=pltpu.CompilerParams(needs_layout_passes=False),
        scratch_types=[
            pltpu.VMEM((T * 5,), _f32),
            pltpu.VMEM((32,), _i32),
            pltpu.VMEM((32,), _i32),
            pltpu.VMEM((256,), _f32),
            pltpu.VMEM((T, CH3), _f32),
            pltpu.SemaphoreType.DMA,
        ],
    )
    return fn(tflat, p4d0, p4d1, p4d2)


def _tc_fused(p0, p1, p2, rows, meta, targets):
    def body(p0_ref, p1_ref, p2_ref, rows_ref, meta_ref, tg_ref, out_ref,
             acc_ref):
        i = pl.program_id(0)

        @pl.when(i == 0)
        def _():
            for s in range(3):
                acc_ref[s] = 0.0

        clip = lambda p: jnp.clip(p, 1e-7, 1.0 - 1e-7)
        sig = jax.nn.sigmoid

        rowc = lax.broadcasted_iota(_i32, (CH3, 3), 0)
        colj = lax.broadcasted_iota(_i32, (CH3, 3), 1)
        sel = (rowc == colj * NCH + 4).astype(_f32)
        for s, pref in enumerate((p0_ref, p1_ref, p2_ref)):
            fs = FS[s]
            x2 = pref[...].reshape(fs * fs, CH3)
            y = lax.dot_general(sel, x2, (((0,), (1,)), ((), ())),
                                preferred_element_type=_f32)
            p = sig(y)
            g = -jnp.log(1.0 - clip(p))
            acc_ref[s] = acc_ref[s] + jnp.sum(g)

        @pl.when(i == B - 1)
        def _():
            tg = tg_ref[...]
            total = jnp.float32(0.0)
            for s in range(3):
                fs = FS[s]
                n = float(B * fs * fs * 3)
                rfull = rows_ref[s]
                m = meta_ref[s]
                win = m[:, 0, :]
                pairw = m[:, 1, :]
                awa = m[:, 2, :]
                aha = m[:, 3, :]
                gxf = m[:, 4, :]
                gyf = m[:, 5, :]
                af = m[:, 6, :]
                a3 = af[:, :, None]
                r = jnp.where(a3 == 0.0, rfull[:, :, 0:NCH],
                              jnp.where(a3 == 1.0, rfull[:, :, NCH:2 * NCH],
                                        rfull[:, :, 2 * NCH:3 * NCH]))
                box0 = tg[:, :, 0] * fs
                box1 = tg[:, :, 1] * fs
                box2 = tg[:, :, 2] * fs
                box3 = tg[:, :, 3] * fs
                ci = tg[:, :, 4].astype(_i32)

                dcx = sig(r[:, :, 0]) + gxf - (box0 - gxf)
                dcy = sig(r[:, :, 1]) + gyf - (box1 - gyf)
                dw = jnp.exp(r[:, :, 2]) * awa - jnp.log(box2 / awa + 1e-6)
                dh = jnp.exp(r[:, :, 3]) * aha - jnp.log(box3 / aha + 1e-6)
                coord = 5.0 * jnp.sum(
                    (dcx * dcx + dcy * dcy + dw * dw + dh * dh) * win) / n

                pc = sig(r[:, :, 4])
                P = jnp.sum(win)
                obj = (jnp.sum(-jnp.log(clip(pc)) * win)
                       + (n - P) * EPS_TERM) / n
                noobj = 0.5 * (acc_ref[s]
                               - jnp.sum(-jnp.log(1.0 - clip(pc)) * win)
                               + P * EPS_TERM) / n

                spc = sig(r[:, :, 5:])
                cellsum = jnp.sum(-jnp.log(1.0 - clip(spc)), axis=2)
                clsidx = lax.broadcasted_iota(_i32, (B, T, NUM_CLASSES), 2)
                oh = clsidx == ci[:, :, None]
                pv = jnp.sum(jnp.where(oh, spc, 0.0), axis=2)
                h = -jnp.log(clip(pv)) + jnp.log(1.0 - clip(pv))
                cls = (jnp.sum(cellsum * win) + jnp.sum(h * pairw)
                       + (n - P) * NUM_CLASSES * EPS_TERM) / (n * NUM_CLASSES)

                total = total + coord + obj + noobj + cls
            out_ref[:, :] = (total / 3.0).reshape(1, 1)

    z4 = lambda i: (i, 0, 0, 0)
    park3 = lambda i: (0, 0, 0)
    return pl.pallas_call(
        body,
        grid=(B,),
        in_specs=[
            pl.BlockSpec((1, FS[0], FS[0], CH3), z4),
            pl.BlockSpec((1, FS[1], FS[1], CH3), z4),
            pl.BlockSpec((1, FS[2], FS[2], CH3), z4),
            pl.BlockSpec((3, B, T, CH3), lambda i: (0, 0, 0, 0)),
            pl.BlockSpec((3, B, 8, T), lambda i: (0, 0, 0, 0)),
            pl.BlockSpec((B, T, 5), park3),
        ],
        out_specs=pl.BlockSpec((1, 1), lambda i: (0, 0)),
        out_shape=jax.ShapeDtypeStruct((1, 1), _f32),
        scratch_shapes=[pltpu.SMEM((3,), _f32)],
    )(p0, p1, p2, rows, meta, targets)


def kernel(pred_s0, pred_s1, pred_s2, targets):
    tflat = targets.reshape(M * 5)
    rows, meta = _sc_assign_gather(tflat, pred_s0, pred_s1, pred_s2)
    meta = meta.reshape(3, B, 8, T)
    out = _tc_fused(pred_s0, pred_s1, pred_s2, rows, meta, targets)
    return out[0, 0]


# speedup vs baseline: 606.0992x; 1.0563x over previous
import functools
import numpy as np
import jax
import jax.numpy as jnp
from jax import lax
from jax.experimental import pallas as pl
from jax.experimental.pallas import tpu as pltpu
from jax.experimental.pallas import tpu_sc as plsc

NUM_CLASSES = 80
IMG_SIZE = 416
_ANCHORS = np.array([[10., 13.], [16., 30.], [33., 23.], [30., 61.],
                     [62., 45.], [59., 119.], [116., 90.], [156., 198.],
                     [373., 326.]], dtype=np.float32)
_MASKS = [[6, 7, 8], [3, 4, 5], [0, 1, 2]]
_STRIDES = [8, 16, 32]
FS = [IMG_SIZE // s for s in _STRIDES]
B, T = 16, 32
M = B * T
NCH = 5 + NUM_CLASSES
CH3 = 3 * NCH
AWH = []
for i in range(3):
    a = _ANCHORS[_MASKS[i]] / float(_STRIDES[i])
    AWH.append(([float(x) for x in a[:, 0]], [float(x) for x in a[:, 1]]))

_E1 = np.float32(1.0) - np.float32(1e-7)
EPS_TERM = float(-np.log(_E1))

_f32 = jnp.float32
_i32 = jnp.int32


def _sc_assign_gather(tflat):
    mesh = plsc.VectorSubcoreMesh(core_axis_name="c", subcore_axis_name="s")

    def body(t_hbm, meta_hbm, traw, cellv, pairv, metav):
        cid = lax.axis_index("c")
        sid = lax.axis_index("s")
        wid = sid * 2 + cid

        iota = lax.iota(_i32, 16)

        def do_unit(scale, b):
            fs = FS[scale]
            aw, ah = AWH[scale]

            pltpu.sync_copy(t_hbm.at[pl.ds(b * (T * 5), T * 5)], traw)

            cells, pairs, gxs, gys, aas = [], [], [], [], []
            for v in range(2):
                sl = pl.ds(16 * v, 16)
                fidx = (iota + 16 * v) * 5
                tx = plsc.load_gather(traw, [fidx])
                ty = plsc.load_gather(traw, [fidx + 1])
                tw = plsc.load_gather(traw, [fidx + 2])
                th = plsc.load_gather(traw, [fidx + 3])
                tc = plsc.load_gather(traw, [fidx + 4])
                box0 = tx * float(fs)
                box1 = ty * float(fs)
                box2 = tw * float(fs)
                box3 = th * float(fs)
                ci = tc.astype(_i32)
                best = None
                aidx = None
                for k in range(3):
                    inter = jnp.minimum(box2, aw[k]) * jnp.minimum(box3, ah[k])
                    union = box2 * box3 + aw[k] * ah[k] - inter
                    iou = inter / (union + 1e-6)
                    if k == 0:
                        best = iou
                        aidx = jnp.zeros((16,), _i32)
                    else:
                        upd = iou > best
                        best = jnp.where(upd, iou, best)
                        aidx = jnp.where(upd, k, aidx)
                gxi = jnp.minimum(box0.astype(_i32), fs - 1)
                gyi = jnp.minimum(box1.astype(_i32), fs - 1)
                cell = ((b * fs + gyi) * fs + gxi) * 3 + aidx
                pair = cell * NUM_CLASSES + ci
                cellv[sl] = cell
                pairv[sl] = pair
                cells.append(cell)
                pairs.append(pair)
                gxs.append(gxi)
                gys.append(gyi)
                aas.append(aidx)
                awa = jnp.where(aidx == 0, aw[0],
                                jnp.where(aidx == 1, aw[1], aw[2]))
                aha = jnp.where(aidx == 0, ah[0],
                                jnp.where(aidx == 1, ah[1], ah[2]))
                metav[pl.ds(2 * 32 + 16 * v, 16)] = awa
                metav[pl.ds(3 * 32 + 16 * v, 16)] = aha
                metav[pl.ds(4 * 32 + 16 * v, 16)] = gxi.astype(_f32)
                metav[pl.ds(5 * 32 + 16 * v, 16)] = gyi.astype(_f32)
                metav[pl.ds(6 * 32 + 16 * v, 16)] = aidx.astype(_f32)
                metav[pl.ds(7 * 32 + 16 * v, 16)] = (
                    gyi * fs + gxi).astype(_f32)

            tvec0 = iota
            tvec1 = iota + 16
            zero16 = jnp.zeros((16,), _i32)

            def shift_body(s, carry):
                h0, h1, q0, q1 = carry
                t2a = jnp.minimum(tvec0 + s, 31)
                va = (tvec0 + s) <= 31
                t2b = jnp.minimum(tvec1 + s, 31)
                vb = (tvec1 + s) <= 31
                c2a = plsc.load_gather(cellv, [t2a])
                c2b = plsc.load_gather(cellv, [t2b])
                p2a = plsc.load_gather(pairv, [t2a])
                p2b = plsc.load_gather(pairv, [t2b])
                h0 = h0 | jnp.where((c2a == cells[0]) & va, 1, zero16)
                h1 = h1 | jnp.where((c2b == cells[1]) & vb, 1, zero16)
                q0 = q0 | jnp.where((p2a == pairs[0]) & va, 1, zero16)
                q1 = q1 | jnp.where((p2b == pairs[1]) & vb, 1, zero16)
                return (h0, h1, q0, q1)

            h0, h1, q0, q1 = lax.fori_loop(
                1, 32, shift_body, (zero16, zero16, zero16, zero16))
            metav[pl.ds(0, 16)] = jnp.where(h0 == 0, 1.0, 0.0).astype(_f32)
            metav[pl.ds(16, 16)] = jnp.where(h1 == 0, 1.0, 0.0).astype(_f32)
            metav[pl.ds(32, 16)] = jnp.where(q0 == 0, 1.0, 0.0).astype(_f32)
            metav[pl.ds(48, 16)] = jnp.where(q1 == 0, 1.0, 0.0).astype(_f32)

            pltpu.sync_copy(metav, meta_hbm.at[scale, b])

        @pl.when(wid < 16)
        def _():
            do_unit(0, wid)
            do_unit(2, wid)

        @pl.when(wid >= 16)
        def _():
            do_unit(1, wid - 16)

    fn = pl.kernel(
        body,
        out_type=jax.ShapeDtypeStruct((3, B, 256), _f32),
        mesh=mesh,
        compiler_params=pltpu.CompilerParams(needs_layout_passes=False),
        scratch_types=[
            pltpu.VMEM((T * 5,), _f32),
            pltpu.VMEM((32,), _i32),
            pltpu.VMEM((32,), _i32),
            pltpu.VMEM((256,), _f32),
        ],
    )
    return fn(tflat)


def _tc_fused(p0, p1, p2, meta, targets):
    def body(p0_ref, p1_ref, p2_ref, ms_ref, meta_ref, tg_ref, out_ref,
             acc_ref, rows_ref):
        i = pl.program_id(0)

        @pl.when(i == 0)
        def _():
            for s in range(3):
                acc_ref[s] = 0.0

        clip = lambda p: jnp.clip(p, 1e-7, 1.0 - 1e-7)
        sig = jax.nn.sigmoid

        rowc = lax.broadcasted_iota(_i32, (CH3, 3), 0)
        colj = lax.broadcasted_iota(_i32, (CH3, 3), 1)
        sel = (rowc == colj * NCH + 4).astype(_f32)
        mb = ms_ref[...]
        for s, pref in enumerate((p0_ref, p1_ref, p2_ref)):
            fs = FS[s]
            x2 = pref[...].reshape(fs * fs, CH3)
            y = lax.dot_general(sel, x2, (((0,), (1,)), ((), ())),
                                preferred_element_type=_f32)
            p = sig(y)
            g = -jnp.log(1.0 - clip(p))
            acc_ref[s] = acc_ref[s] + jnp.sum(g)
            ri = mb[s, 0, 7, :].astype(_i32)
            oh = (lax.broadcasted_iota(_i32, (T, fs * fs), 1)
                  == ri[:, None]).astype(_f32)
            rblk = lax.dot_general(oh, x2, (((1,), (0,)), ((), ())),
                                   preferred_element_type=_f32)
            rows_ref[s, pl.ds(i, 1)] = rblk[None]

        @pl.when(i == B - 1)
        def _():
            tg = tg_ref[...]
            total = jnp.float32(0.0)
            for s in range(3):
                fs = FS[s]
                n = float(B * fs * fs * 3)
                rfull = rows_ref[s]
                m = meta_ref[s]
                win = m[:, 0, :]
                pairw = m[:, 1, :]
                awa = m[:, 2, :]
                aha = m[:, 3, :]
                gxf = m[:, 4, :]
                gyf = m[:, 5, :]
                af = m[:, 6, :]
                a3 = af[:, :, None]
                r = jnp.where(a3 == 0.0, rfull[:, :, 0:NCH],
                              jnp.where(a3 == 1.0, rfull[:, :, NCH:2 * NCH],
                                        rfull[:, :, 2 * NCH:3 * NCH]))
                box0 = tg[:, :, 0] * fs
                box1 = tg[:, :, 1] * fs
                box2 = tg[:, :, 2] * fs
                box3 = tg[:, :, 3] * fs
                ci = tg[:, :, 4].astype(_i32)

                dcx = sig(r[:, :, 0]) + gxf - (box0 - gxf)
                dcy = sig(r[:, :, 1]) + gyf - (box1 - gyf)
                dw = jnp.exp(r[:, :, 2]) * awa - jnp.log(box2 / awa + 1e-6)
                dh = jnp.exp(r[:, :, 3]) * aha - jnp.log(box3 / aha + 1e-6)
                coord = 5.0 * jnp.sum(
                    (dcx * dcx + dcy * dcy + dw * dw + dh * dh) * win) / n

                pc = sig(r[:, :, 4])
                P = jnp.sum(win)
                obj = (jnp.sum(-jnp.log(clip(pc)) * win)
                       + (n - P) * EPS_TERM) / n
                noobj = 0.5 * (acc_ref[s]
                               - jnp.sum(-jnp.log(1.0 - clip(pc)) * win)
                               + P * EPS_TERM) / n

                spc = sig(r[:, :, 5:])
                cellsum = jnp.sum(-jnp.log(1.0 - clip(spc)), axis=2)
                clsidx = lax.broadcasted_iota(_i32, (B, T, NUM_CLASSES), 2)
                oh = clsidx == ci[:, :, None]
                pv = jnp.sum(jnp.where(oh, spc, 0.0), axis=2)
                h = -jnp.log(clip(pv)) + jnp.log(1.0 - clip(pv))
                cls = (jnp.sum(cellsum * win) + jnp.sum(h * pairw)
                       + (n - P) * NUM_CLASSES * EPS_TERM) / (n * NUM_CLASSES)

                total = total + coord + obj + noobj + cls
            out_ref[:, :] = (total / 3.0).reshape(1, 1)

    z4 = lambda i: (i, 0, 0, 0)
    park3 = lambda i: (0, 0, 0)
    return pl.pallas_call(
        body,
        grid=(B,),
        in_specs=[
            pl.BlockSpec((1, FS[0], FS[0], CH3), z4),
            pl.BlockSpec((1, FS[1], FS[1], CH3), z4),
            pl.BlockSpec((1, FS[2], FS[2], CH3), z4),
            pl.BlockSpec((3, 1, 8, T), lambda i: (0, i, 0, 0)),
            pl.BlockSpec((3, B, 8, T), lambda i: (0, 0, 0, 0)),
            pl.BlockSpec((B, T, 5), park3),
        ],
        out_specs=pl.BlockSpec((1, 1), lambda i: (0, 0)),
        out_shape=jax.ShapeDtypeStruct((1, 1), _f32),
        scratch_shapes=[pltpu.SMEM((3,), _f32),
                        pltpu.VMEM((3, B, T, CH3), _f32)],
    )(p0, p1, p2, meta, meta, targets)


def kernel(pred_s0, pred_s1, pred_s2, targets):
    tflat = targets.reshape(M * 5)
    meta = _sc_assign_gather(tflat)
    meta = meta.reshape(3, B, 8, T)
    out = _tc_fused(pred_s0, pred_s1, pred_s2, meta, targets)
    return out[0, 0]
